# baseline (speedup 1.0000x reference)
import functools

import jax
import jax.numpy as jnp
from jax import lax
from jax.experimental import pallas as pl
from jax.experimental.pallas import tpu as pltpu
from jax.experimental.pallas import tpu_sc as plsc

N = 50000
HID = 64
G = 256
EPS = 1e-5

NC, NS, L = 2, 16, 16
NW = NC * NS

E_PAD = 802816
EPT_A = E_PAD // NW
EPT_B = E_PAD // NS
STAGE = 1792
CHUNK = 32
DEG_PAD = 50176
HALF = N // NC
ROWS_PT = 1568
ACC_ROWS = NS * ROWS_PT
N_PAD = 50176
DUMP_FILL = 2**30

@functools.cache
def _mesh():
    return plsc.VectorSubcoreMesh(
        core_axis_name="c", subcore_axis_name="s",
        num_cores=NC, num_subcores=NS)


def _deg_body(dst_hbm, zeros_hbm, deg_out, idx_v, deg_v):
    c = lax.axis_index("c")
    s = lax.axis_index("s")
    wid = c * NS + s
    pltpu.sync_copy(zeros_hbm, deg_v)
    base = wid * EPT_A
    lanes = lax.iota(jnp.int32, L)
    ones = jnp.ones((L,), jnp.float32)

    def stage_body(k, _):
        pltpu.sync_copy(dst_hbm.at[pl.ds(base + k * STAGE, STAGE)], idx_v)

        def vec_body(j, _):
            d = idx_v[pl.ds(j * L, L)]
            dloc = jnp.where(d < N, d, N + lanes)
            plsc.addupdate_scatter(deg_v, [dloc], ones)
            return 0

        return lax.fori_loop(0, STAGE // L, vec_body, 0)

    lax.fori_loop(0, EPT_A // STAGE, stage_body, 0)
    pltpu.sync_copy(deg_v, deg_out.at[wid])


def _deg_call(dst_p, zeros_deg):
    f = pl.kernel(
        _deg_body,
        out_type=jax.ShapeDtypeStruct((NW, DEG_PAD), jnp.float32),
        mesh=_mesh(),
        compiler_params=pltpu.CompilerParams(needs_layout_passes=False, use_tc_tiling_on_sc=False),
        scratch_types=[
            pltpu.VMEM((STAGE,), jnp.int32),
            pltpu.VMEM((DEG_PAD,), jnp.float32),
        ],
    )
    return f(dst_p, zeros_deg)


NBUF = 8


def _agg_body(src_hbm, dst_hbm, hs_hbm, zeros_hbm, out_hbm,
              sidx_v, didx_v, scat_v, rows_v, acc_sh, *sems):
    gsems = sems[:NBUF]
    ssems = sems[NBUF:]
    c = lax.axis_index("c")
    s = lax.axis_index("s")
    my_rows = pl.ds(s * ROWS_PT, ROWS_PT)
    pltpu.sync_copy(zeros_hbm, acc_sh.at[my_rows])
    plsc.subcore_barrier()

    base = s * EPT_B
    half_lo = c * HALF
    lanes = lax.iota(jnp.int32, L)
    n_chunks = STAGE // CHUNK

    def compute_scat(b, j):
        def vec_body(v, _):
            dv = didx_v[pl.ds(j * CHUNK + v * L, L)]
            t = dv - half_lo
            ok = (t >= 0) & (t < HALF)
            spread = HALF + (v % 4) * L
            scat_v[b, pl.ds(v * L, L)] = jnp.where(ok, t, spread + lanes)
            return 0

        lax.fori_loop(0, CHUNK // L, vec_body, 0)

    def fire_gather(b, j):
        return pltpu.async_copy(
            hs_hbm.at[sidx_v.at[pl.ds(j * CHUNK, CHUNK)]], rows_v.at[b],
            gsems[b])

    def stage_body(k, _):
        off = base + k * STAGE
        pltpu.sync_copy(src_hbm.at[pl.ds(off, STAGE)], sidx_v)
        pltpu.sync_copy(dst_hbm.at[pl.ds(off, STAGE)], didx_v)
        for b in range(NBUF):
            compute_scat(b, b)
            fire_gather(b, b)

        def ring_body(j0, _):
            for b in range(NBUF):
                j = j0 * NBUF + b
                pltpu.make_async_copy(
                    hs_hbm.at[sidx_v.at[pl.ds(0, CHUNK)]], rows_v.at[b],
                    gsems[b]).wait()
                pltpu.async_copy(rows_v.at[b], acc_sh.at[scat_v.at[b]],
                                 ssems[b], add=True).wait()
                jn = j + NBUF

                @pl.when(jn < n_chunks)
                def _():
                    compute_scat(b, jn)
                    fire_gather(b, jn)

            return 0

        return lax.fori_loop(0, n_chunks // NBUF, ring_body, 0,
                             unroll=False)

    lax.fori_loop(0, EPT_B // STAGE, stage_body, 0)
    plsc.subcore_barrier()
    pltpu.sync_copy(acc_sh.at[my_rows],
                    out_hbm.at[pl.ds(c * ACC_ROWS + s * ROWS_PT, ROWS_PT)])


def _agg_call(src_p, dst_p, hs, zeros_acc):
    f = pl.kernel(
        _agg_body,
        out_type=jax.ShapeDtypeStruct((NC * ACC_ROWS, HID), jnp.float32),
        mesh=_mesh(),
        compiler_params=pltpu.CompilerParams(needs_layout_passes=False, use_tc_tiling_on_sc=False),
        scratch_types=[
            pltpu.VMEM((STAGE,), jnp.int32),
            pltpu.VMEM((STAGE,), jnp.int32),
            pltpu.VMEM((NBUF, CHUNK), jnp.int32),
            pltpu.VMEM((NBUF, CHUNK, HID), jnp.float32),
            pltpu.VMEM_SHARED((ACC_ROWS, HID), jnp.float32),
        ] + [pltpu.SemaphoreType.DMA] * (2 * NBUF),
    )
    return f(src_p, dst_p, hs, zeros_acc)


def _extract(starts_v, g):
    b = (g // L) * L
    vec = starts_v[pl.ds(b, L)]
    m = lax.iota(jnp.int32, L) == (g - b)
    return jnp.sum(jnp.where(m, vec, 0))


def _pool_body(h3_hbm, starts_hbm, sums_hbm, maxs_hbm,
               starts_v, buf_v, sum8_v, max8_v):
    c = lax.axis_index("c")
    s = lax.axis_index("s")
    wid = c * NS + s
    g0 = wid * (G // NW)
    pltpu.sync_copy(starts_hbm, starts_v)

    zero = jnp.zeros((L,), jnp.float32)
    ninf = jnp.full((L,), -jnp.inf, jnp.float32)

    for k in range(G // NW):
        g = g0 + k
        sg = _extract(starts_v, g)
        eg = _extract(starts_v, g + 1)
        nch = (eg - sg + CHUNK - 1) // CHUNK

        def chunk_body(ci, carry, sg=sg, eg=eg):
            st = sg + ci * CHUNK
            pltpu.sync_copy(h3_hbm.at[pl.ds(st, CHUNK)], buf_v)
            cnt = jnp.minimum(CHUNK, eg - st)

            def row_body(r, cr):
                s0, s1, s2, s3, m0, m1, m2, m3 = cr
                x0 = buf_v[r, pl.ds(0, L)]
                x1 = buf_v[r, pl.ds(L, L)]
                x2 = buf_v[r, pl.ds(2 * L, L)]
                x3 = buf_v[r, pl.ds(3 * L, L)]
                return (s0 + x0, s1 + x1, s2 + x2, s3 + x3,
                        jnp.maximum(m0, x0), jnp.maximum(m1, x1),
                        jnp.maximum(m2, x2), jnp.maximum(m3, x3))

            return lax.fori_loop(0, cnt, row_body, carry)

        res = lax.fori_loop(0, nch, chunk_body,
                            (zero, zero, zero, zero, ninf, ninf, ninf, ninf))
        for j in range(4):
            sum8_v[k, pl.ds(j * L, L)] = res[j]
            max8_v[k, pl.ds(j * L, L)] = res[4 + j]

    pltpu.sync_copy(sum8_v, sums_hbm.at[pl.ds(g0, G // NW)])
    pltpu.sync_copy(max8_v, maxs_hbm.at[pl.ds(g0, G // NW)])


def _pool_call(h3, starts1d):
    f = pl.kernel(
        _pool_body,
        out_type=(
            jax.ShapeDtypeStruct((G, HID), jnp.float32),
            jax.ShapeDtypeStruct((G, HID), jnp.float32),
        ),
        mesh=_mesh(),
        compiler_params=pltpu.CompilerParams(needs_layout_passes=False, use_tc_tiling_on_sc=False),
        scratch_types=[
            pltpu.VMEM((272,), jnp.int32),
            pltpu.VMEM((CHUNK, HID), jnp.float32),
            pltpu.VMEM((G // NW, HID), jnp.float32),
            pltpu.VMEM((G // NW, HID), jnp.float32),
        ],
    )
    return f(h3, starts1d)


BLK = 1000
NB = N // BLK


def _t1_body(x_ref, w_ref, degp_ref, hs_ref, dinv_ref):
    i = pl.program_id(0)
    deg = jnp.sum(degp_ref[:, pl.ds(i * BLK3, BLK3)], axis=0) + 1.0
    dinv = lax.rsqrt(deg)
    h = jnp.dot(x_ref[...], w_ref[...], preferred_element_type=jnp.float32)
    hs_ref[...] = h * dinv[:, None]
    dinv_ref[...] = dinv[:, None]


def _t1_call(x, W1, degp):
    d_in = x.shape[1]
    return pl.pallas_call(
        _t1_body,
        grid=(NB3,),
        in_specs=[
            pl.BlockSpec((BLK3, d_in), lambda i: (i, 0)),
            pl.BlockSpec((d_in, HID), lambda i: (0, 0)),
            pl.BlockSpec((NW, DEG_PAD), lambda i: (0, 0)),
        ],
        out_specs=[
            pl.BlockSpec((BLK3, HID), lambda i: (i, 0)),
            pl.BlockSpec((BLK3, 1), lambda i: (i, 0)),
        ],
        out_shape=[
            jax.ShapeDtypeStruct((N_PAD, HID), jnp.float32),
            jax.ShapeDtypeStruct((N_PAD, 1), jnp.float32),
        ],
    )(x, W1, degp)


def _t2a_body(agg_ref, hs_ref, dinv_ref, b_ref, z_ref, st_ref):
    z = dinv_ref[...] * (agg_ref[0] + hs_ref[...]) + b_ref[...]
    z_ref[...] = z
    st_ref[...] = jnp.stack([jnp.sum(z, axis=0), jnp.sum(z * z, axis=0)])[None]


def _t2a_call(agg3, hs, dinv, b):
    return pl.pallas_call(
        _t2a_body,
        grid=(NB,),
        in_specs=[
            pl.BlockSpec((1, BLK, HID), lambda i: (i // 25, i % 25, 0)),
            pl.BlockSpec((BLK, HID), lambda i: (i, 0)),
            pl.BlockSpec((BLK, 1), lambda i: (i, 0)),
            pl.BlockSpec((1, HID), lambda i: (0, 0)),
        ],
        out_specs=[
            pl.BlockSpec((BLK, HID), lambda i: (i, 0)),
            pl.BlockSpec((1, 2, HID), lambda i: (i, 0, 0)),
        ],
        out_shape=[
            jax.ShapeDtypeStruct((N, HID), jnp.float32),
            jax.ShapeDtypeStruct((NB, 2, HID), jnp.float32),
        ],
    )(agg3, hs, dinv, b)


def _bn_relu(z, st, g, be):
    m = jnp.sum(st[:, 0, :], axis=0) * (1.0 / N)
    ex2 = jnp.sum(st[:, 1, :], axis=0) * (1.0 / N)
    istd = lax.rsqrt(ex2 - m * m + EPS)
    return jnp.maximum((z - m) * istd * g + be, 0.0)


def _t2b_body(z_ref, st_ref, dinv_ref, g_ref, be_ref, w_ref, out_ref):
    h = _bn_relu(z_ref[...], st_ref[...], g_ref[...], be_ref[...])
    out_ref[...] = (jnp.dot(h, w_ref[...], preferred_element_type=jnp.float32)
                    * dinv_ref[...])


def _t2b_call(z, st, dinv, g, be, Wn):
    return pl.pallas_call(
        _t2b_body,
        grid=(NB,),
        in_specs=[
            pl.BlockSpec((BLK, HID), lambda i: (i, 0)),
            pl.BlockSpec((NB, 2, HID), lambda i: (0, 0, 0)),
            pl.BlockSpec((BLK, 1), lambda i: (i, 0)),
            pl.BlockSpec((1, HID), lambda i: (0, 0)),
            pl.BlockSpec((1, HID), lambda i: (0, 0)),
            pl.BlockSpec((HID, HID), lambda i: (0, 0)),
        ],
        out_specs=pl.BlockSpec((BLK, HID), lambda i: (i, 0)),
        out_shape=jax.ShapeDtypeStruct((N_PAD, HID), jnp.float32),
    )(z, st, dinv, g, be, Wn)


BLK3 = 1024
NB3 = N_PAD // BLK3


def _t2b3_body(z_ref, st_ref, g_ref, be_ref, out_ref):
    out_ref[...] = _bn_relu(z_ref[...], st_ref[...], g_ref[...], be_ref[...])


def _t2b3_call(z, st, g, be):
    return pl.pallas_call(
        _t2b3_body,
        grid=(NB3,),
        in_specs=[
            pl.BlockSpec((BLK3, HID), lambda i: (i, 0)),
            pl.BlockSpec((NB, 2, HID), lambda i: (0, 0, 0)),
            pl.BlockSpec((1, HID), lambda i: (0, 0)),
            pl.BlockSpec((1, HID), lambda i: (0, 0)),
        ],
        out_specs=pl.BlockSpec((BLK3, HID), lambda i: (i, 0)),
        out_shape=jax.ShapeDtypeStruct((N_PAD, HID), jnp.float32),
    )(z, st, g, be)


GW = 272


def _t3_body(b_ref, starts_ref, counts_ref, acc):
    i = pl.program_id(0)

    @pl.when(i == 0)
    def _():
        acc[...] = jnp.zeros((GW, 1), jnp.float32)

    ids = b_ref[...][0]
    gids = lax.broadcasted_iota(jnp.int32, (GW, BLK), 0)
    oh = (gids == jnp.broadcast_to(ids, (GW, BLK))).astype(jnp.float32)
    acc[...] += jnp.sum(oh, axis=1, keepdims=True)

    @pl.when(i == NB - 1)
    def _():
        cts = acc[...]
        ii = lax.broadcasted_iota(jnp.int32, (GW, GW), 0)
        jj = lax.broadcasted_iota(jnp.int32, (GW, GW), 1)
        tri = (jj < ii).astype(jnp.float32)
        starts_ref[...] = jnp.dot(
            tri, cts, preferred_element_type=jnp.float32).astype(jnp.int32)
        counts_ref[...] = cts


def _t3_call(batch_r):
    return pl.pallas_call(
        _t3_body,
        grid=(NB,),
        in_specs=[pl.BlockSpec((1, 1, BLK), lambda i: (i, 0, 0))],
        out_specs=[
            pl.BlockSpec((GW, 1), lambda i: (0, 0)),
            pl.BlockSpec((GW, 1), lambda i: (0, 0)),
        ],
        out_shape=[
            jax.ShapeDtypeStruct((GW, 1), jnp.int32),
            jax.ShapeDtypeStruct((GW, 1), jnp.float32),
        ],
        scratch_shapes=[pltpu.VMEM((GW, 1), jnp.float32)],
    )(batch_r)


def _t4_body(sums_ref, maxs_ref, counts_ref, wp1_ref, bp1_ref, wp2_ref,
             bp2_ref, out_ref):
    cnt = counts_ref[...][:G]
    mean = sums_ref[...] / jnp.maximum(cnt, 1.0)
    hc = jnp.concatenate([mean, maxs_ref[...]], axis=1)
    h = jnp.maximum(
        jnp.dot(hc, wp1_ref[...], preferred_element_type=jnp.float32)
        + bp1_ref[...], 0.0)
    out_ref[...] = (jnp.dot(h, wp2_ref[...], preferred_element_type=jnp.float32)
                    + bp2_ref[...])


def _t4_call(sums, maxs, counts, Wp1, bp1, Wp2, bp2):
    return pl.pallas_call(
        _t4_body,
        in_specs=[
            pl.BlockSpec((G, HID), lambda: (0, 0)),
            pl.BlockSpec((G, HID), lambda: (0, 0)),
            pl.BlockSpec((GW, 1), lambda: (0, 0)),
            pl.BlockSpec((2 * HID, HID), lambda: (0, 0)),
            pl.BlockSpec((1, HID), lambda: (0, 0)),
            pl.BlockSpec((HID, 3), lambda: (0, 0)),
            pl.BlockSpec((1, 3), lambda: (0, 0)),
        ],
        out_specs=pl.BlockSpec((G, 3), lambda: (0, 0)),
        out_shape=jax.ShapeDtypeStruct((G, 3), jnp.float32),
    )(sums, maxs, counts, Wp1, bp1, Wp2, bp2)


def kernel(x, edge_index, batch, W1, b1, W2, b2, W3, b3, g1, be1, g2, be2,
           g3, be3, Wp1, bp1, Wp2, bp2):
    e = edge_index.shape[1]
    pad = E_PAD - e
    src_p = jnp.concatenate([edge_index[0], jnp.zeros((pad,), jnp.int32)])
    dst_p = jnp.concatenate(
        [edge_index[1], jnp.full((pad,), DUMP_FILL, jnp.int32)])

    zeros_deg = jnp.zeros((DEG_PAD,), jnp.float32)
    zeros_acc = jnp.zeros((ROWS_PT, HID), jnp.float32)

    degp = _deg_call(dst_p, zeros_deg)
    starts_c, counts_c = _t3_call(batch.reshape(NB, 1, BLK))
    starts1d = starts_c.reshape(GW)

    hs, dinv = _t1_call(x, W1, degp)

    layers = ((b1, g1, be1, W2), (b2, g2, be2, W3), (b3, g3, be3, None))
    h3 = None
    for b, g, be, Wn in layers:
        agg = _agg_call(src_p, dst_p, hs, zeros_acc)
        agg3 = agg.reshape(NC, ACC_ROWS, HID)
        z, st = _t2a_call(agg3, hs, dinv, b.reshape(1, HID))
        if Wn is not None:
            hs = _t2b_call(z, st, dinv, g.reshape(1, HID), be.reshape(1, HID),
                           Wn)
        else:
            h3 = _t2b3_call(z, st, g.reshape(1, HID), be.reshape(1, HID))

    sums, maxs = _pool_call(h3, starts1d)
    return _t4_call(sums, maxs, counts_c, Wp1, bp1.reshape(1, HID), Wp2,
                    bp2.reshape(1, 3))


# speedup vs baseline: 1.0543x; 1.0543x over previous
import functools

import jax
import jax.numpy as jnp
from jax import lax
from jax.experimental import pallas as pl
from jax.experimental.pallas import tpu as pltpu
from jax.experimental.pallas import tpu_sc as plsc

N = 50000
HID = 64
G = 256
EPS = 1e-5

NC, NS, L = 2, 16, 16
NW = NC * NS

E_PAD = 802816
EPT_A = E_PAD // NW
EPT_B = E_PAD // NS
STAGE = 1792
CHUNK = 64
DEG_PAD = 50176
HALF = N // NC
ROWS_PT = 1568
ACC_ROWS = NS * ROWS_PT
N_PAD = 50176
DUMP_FILL = 2**30

@functools.cache
def _mesh():
    return plsc.VectorSubcoreMesh(
        core_axis_name="c", subcore_axis_name="s",
        num_cores=NC, num_subcores=NS)


def _deg_body(dst_hbm, zeros_hbm, deg_out, idx_v, deg_v):
    c = lax.axis_index("c")
    s = lax.axis_index("s")
    wid = c * NS + s
    pltpu.sync_copy(zeros_hbm, deg_v)
    base = wid * EPT_A
    lanes = lax.iota(jnp.int32, L)
    ones = jnp.ones((L,), jnp.float32)

    def stage_body(k, _):
        pltpu.sync_copy(dst_hbm.at[pl.ds(base + k * STAGE, STAGE)], idx_v)

        def vec_body(j, _):
            d = idx_v[pl.ds(j * L, L)]
            dloc = jnp.where(d < N, d, N + lanes)
            plsc.addupdate_scatter(deg_v, [dloc], ones)
            return 0

        return lax.fori_loop(0, STAGE // L, vec_body, 0)

    lax.fori_loop(0, EPT_A // STAGE, stage_body, 0)
    pltpu.sync_copy(deg_v, deg_out.at[wid])


def _deg_call(dst_p, zeros_deg):
    f = pl.kernel(
        _deg_body,
        out_type=jax.ShapeDtypeStruct((NW, DEG_PAD), jnp.float32),
        mesh=_mesh(),
        compiler_params=pltpu.CompilerParams(needs_layout_passes=False, use_tc_tiling_on_sc=False),
        scratch_types=[
            pltpu.VMEM((STAGE,), jnp.int32),
            pltpu.VMEM((DEG_PAD,), jnp.float32),
        ],
    )
    return f(dst_p, zeros_deg)


NBUF = 4


def _agg_body(src_hbm, dst_hbm, hs_hbm, zeros_hbm, out_hbm,
              sidx_v, didx_v, scat_v, rows_v, acc_sh, *sems):
    gsems = sems[:NBUF]
    ssems = sems[NBUF:]
    c = lax.axis_index("c")
    s = lax.axis_index("s")
    my_rows = pl.ds(s * ROWS_PT, ROWS_PT)
    pltpu.sync_copy(zeros_hbm, acc_sh.at[my_rows])
    plsc.subcore_barrier()

    base = s * EPT_B
    half_lo = c * HALF
    lanes = lax.iota(jnp.int32, L)
    n_chunks = STAGE // CHUNK

    def compute_scat(b, j):
        def vec_body(v, _):
            dv = didx_v[pl.ds(j * CHUNK + v * L, L)]
            t = dv - half_lo
            ok = (t >= 0) & (t < HALF)
            spread = HALF + (v % 4) * L
            scat_v[b, pl.ds(v * L, L)] = jnp.where(ok, t, spread + lanes)
            return 0

        lax.fori_loop(0, CHUNK // L, vec_body, 0)

    def fire_gather(b, j):
        return pltpu.async_copy(
            hs_hbm.at[sidx_v.at[pl.ds(j * CHUNK, CHUNK)]], rows_v.at[b],
            gsems[b])

    def stage_body(k, _):
        off = base + k * STAGE
        pltpu.sync_copy(src_hbm.at[pl.ds(off, STAGE)], sidx_v)
        pltpu.sync_copy(dst_hbm.at[pl.ds(off, STAGE)], didx_v)
        for b in range(NBUF):
            compute_scat(b, b)
            fire_gather(b, b)

        def ring_body(j0, _):
            for b in range(NBUF):
                j = j0 * NBUF + b
                pltpu.make_async_copy(
                    hs_hbm.at[sidx_v.at[pl.ds(0, CHUNK)]], rows_v.at[b],
                    gsems[b]).wait()
                pltpu.async_copy(rows_v.at[b], acc_sh.at[scat_v.at[b]],
                                 ssems[b], add=True).wait()
                jn = j + NBUF

                @pl.when(jn < n_chunks)
                def _():
                    compute_scat(b, jn)
                    fire_gather(b, jn)

            return 0

        return lax.fori_loop(0, n_chunks // NBUF, ring_body, 0,
                             unroll=False)

    lax.fori_loop(0, EPT_B // STAGE, stage_body, 0)
    plsc.subcore_barrier()
    pltpu.sync_copy(acc_sh.at[my_rows],
                    out_hbm.at[pl.ds(c * ACC_ROWS + s * ROWS_PT, ROWS_PT)])


def _agg_call(src_p, dst_p, hs, zeros_acc):
    f = pl.kernel(
        _agg_body,
        out_type=jax.ShapeDtypeStruct((NC * ACC_ROWS, HID), jnp.float32),
        mesh=_mesh(),
        compiler_params=pltpu.CompilerParams(needs_layout_passes=False, use_tc_tiling_on_sc=False),
        scratch_types=[
            pltpu.VMEM((STAGE,), jnp.int32),
            pltpu.VMEM((STAGE,), jnp.int32),
            pltpu.VMEM((NBUF, CHUNK), jnp.int32),
            pltpu.VMEM((NBUF, CHUNK, HID), jnp.float32),
            pltpu.VMEM_SHARED((ACC_ROWS, HID), jnp.float32),
        ] + [pltpu.SemaphoreType.DMA] * (2 * NBUF),
    )
    return f(src_p, dst_p, hs, zeros_acc)


def _extract(starts_v, g):
    b = (g // L) * L
    vec = starts_v[pl.ds(b, L)]
    m = lax.iota(jnp.int32, L) == (g - b)
    return jnp.sum(jnp.where(m, vec, 0))


def _pool_body(h3_hbm, starts_hbm, sums_hbm, maxs_hbm,
               starts_v, buf_v, sum8_v, max8_v):
    c = lax.axis_index("c")
    s = lax.axis_index("s")
    wid = c * NS + s
    g0 = wid * (G // NW)
    pltpu.sync_copy(starts_hbm, starts_v)

    zero = jnp.zeros((L,), jnp.float32)
    ninf = jnp.full((L,), -jnp.inf, jnp.float32)

    for k in range(G // NW):
        g = g0 + k
        sg = _extract(starts_v, g)
        eg = _extract(starts_v, g + 1)
        nch = (eg - sg + CHUNK - 1) // CHUNK

        def chunk_body(ci, carry, sg=sg, eg=eg):
            st = sg + ci * CHUNK
            pltpu.sync_copy(h3_hbm.at[pl.ds(st, CHUNK)], buf_v)
            cnt = jnp.minimum(CHUNK, eg - st)

            def row_body(r, cr):
                s0, s1, s2, s3, m0, m1, m2, m3 = cr
                x0 = buf_v[r, pl.ds(0, L)]
                x1 = buf_v[r, pl.ds(L, L)]
                x2 = buf_v[r, pl.ds(2 * L, L)]
                x3 = buf_v[r, pl.ds(3 * L, L)]
                return (s0 + x0, s1 + x1, s2 + x2, s3 + x3,
                        jnp.maximum(m0, x0), jnp.maximum(m1, x1),
                        jnp.maximum(m2, x2), jnp.maximum(m3, x3))

            return lax.fori_loop(0, cnt, row_body, carry)

        res = lax.fori_loop(0, nch, chunk_body,
                            (zero, zero, zero, zero, ninf, ninf, ninf, ninf))
        for j in range(4):
            sum8_v[k, pl.ds(j * L, L)] = res[j]
            max8_v[k, pl.ds(j * L, L)] = res[4 + j]

    pltpu.sync_copy(sum8_v, sums_hbm.at[pl.ds(g0, G // NW)])
    pltpu.sync_copy(max8_v, maxs_hbm.at[pl.ds(g0, G // NW)])


def _pool_call(h3, starts1d):
    f = pl.kernel(
        _pool_body,
        out_type=(
            jax.ShapeDtypeStruct((G, HID), jnp.float32),
            jax.ShapeDtypeStruct((G, HID), jnp.float32),
        ),
        mesh=_mesh(),
        compiler_params=pltpu.CompilerParams(needs_layout_passes=False, use_tc_tiling_on_sc=False),
        scratch_types=[
            pltpu.VMEM((272,), jnp.int32),
            pltpu.VMEM((CHUNK, HID), jnp.float32),
            pltpu.VMEM((G // NW, HID), jnp.float32),
            pltpu.VMEM((G // NW, HID), jnp.float32),
        ],
    )
    return f(h3, starts1d)


BLK = 1000
NB = N // BLK


def _t1_body(x_ref, w_ref, degp_ref, hs_ref, dinv_ref):
    i = pl.program_id(0)
    deg = jnp.sum(degp_ref[:, pl.ds(i * BLK3, BLK3)], axis=0) + 1.0
    dinv = lax.rsqrt(deg)
    h = jnp.dot(x_ref[...], w_ref[...], preferred_element_type=jnp.float32)
    hs_ref[...] = h * dinv[:, None]
    dinv_ref[...] = dinv[:, None]


def _t1_call(x, W1, degp):
    d_in = x.shape[1]
    return pl.pallas_call(
        _t1_body,
        grid=(NB3,),
        in_specs=[
            pl.BlockSpec((BLK3, d_in), lambda i: (i, 0)),
            pl.BlockSpec((d_in, HID), lambda i: (0, 0)),
            pl.BlockSpec((NW, DEG_PAD), lambda i: (0, 0)),
        ],
        out_specs=[
            pl.BlockSpec((BLK3, HID), lambda i: (i, 0)),
            pl.BlockSpec((BLK3, 1), lambda i: (i, 0)),
        ],
        out_shape=[
            jax.ShapeDtypeStruct((N_PAD, HID), jnp.float32),
            jax.ShapeDtypeStruct((N_PAD, 1), jnp.float32),
        ],
    )(x, W1, degp)


def _t2a_body(agg_ref, hs_ref, dinv_ref, b_ref, z_ref, st_ref):
    z = dinv_ref[...] * (agg_ref[0] + hs_ref[...]) + b_ref[...]
    z_ref[...] = z
    st_ref[...] = jnp.stack([jnp.sum(z, axis=0), jnp.sum(z * z, axis=0)])[None]


def _t2a_call(agg3, hs, dinv, b):
    return pl.pallas_call(
        _t2a_body,
        grid=(NB,),
        in_specs=[
            pl.BlockSpec((1, BLK, HID), lambda i: (i // 25, i % 25, 0)),
            pl.BlockSpec((BLK, HID), lambda i: (i, 0)),
            pl.BlockSpec((BLK, 1), lambda i: (i, 0)),
            pl.BlockSpec((1, HID), lambda i: (0, 0)),
        ],
        out_specs=[
            pl.BlockSpec((BLK, HID), lambda i: (i, 0)),
            pl.BlockSpec((1, 2, HID), lambda i: (i, 0, 0)),
        ],
        out_shape=[
            jax.ShapeDtypeStruct((N, HID), jnp.float32),
            jax.ShapeDtypeStruct((NB, 2, HID), jnp.float32),
        ],
    )(agg3, hs, dinv, b)


def _bn_relu(z, st, g, be):
    m = jnp.sum(st[:, 0, :], axis=0) * (1.0 / N)
    ex2 = jnp.sum(st[:, 1, :], axis=0) * (1.0 / N)
    istd = lax.rsqrt(ex2 - m * m + EPS)
    return jnp.maximum((z - m) * istd * g + be, 0.0)


def _t2b_body(z_ref, st_ref, dinv_ref, g_ref, be_ref, w_ref, out_ref):
    h = _bn_relu(z_ref[...], st_ref[...], g_ref[...], be_ref[...])
    out_ref[...] = (jnp.dot(h, w_ref[...], preferred_element_type=jnp.float32)
                    * dinv_ref[...])


def _t2b_call(z, st, dinv, g, be, Wn):
    return pl.pallas_call(
        _t2b_body,
        grid=(NB,),
        in_specs=[
            pl.BlockSpec((BLK, HID), lambda i: (i, 0)),
            pl.BlockSpec((NB, 2, HID), lambda i: (0, 0, 0)),
            pl.BlockSpec((BLK, 1), lambda i: (i, 0)),
            pl.BlockSpec((1, HID), lambda i: (0, 0)),
            pl.BlockSpec((1, HID), lambda i: (0, 0)),
            pl.BlockSpec((HID, HID), lambda i: (0, 0)),
        ],
        out_specs=pl.BlockSpec((BLK, HID), lambda i: (i, 0)),
        out_shape=jax.ShapeDtypeStruct((N_PAD, HID), jnp.float32),
    )(z, st, dinv, g, be, Wn)


BLK3 = 1024
NB3 = N_PAD // BLK3


def _t2b3_body(z_ref, st_ref, g_ref, be_ref, out_ref):
    out_ref[...] = _bn_relu(z_ref[...], st_ref[...], g_ref[...], be_ref[...])


def _t2b3_call(z, st, g, be):
    return pl.pallas_call(
        _t2b3_body,
        grid=(NB3,),
        in_specs=[
            pl.BlockSpec((BLK3, HID), lambda i: (i, 0)),
            pl.BlockSpec((NB, 2, HID), lambda i: (0, 0, 0)),
            pl.BlockSpec((1, HID), lambda i: (0, 0)),
            pl.BlockSpec((1, HID), lambda i: (0, 0)),
        ],
        out_specs=pl.BlockSpec((BLK3, HID), lambda i: (i, 0)),
        out_shape=jax.ShapeDtypeStruct((N_PAD, HID), jnp.float32),
    )(z, st, g, be)


GW = 272


def _t3_body(b_ref, starts_ref, counts_ref, acc):
    i = pl.program_id(0)

    @pl.when(i == 0)
    def _():
        acc[...] = jnp.zeros((GW, 1), jnp.float32)

    ids = b_ref[...][0]
    gids = lax.broadcasted_iota(jnp.int32, (GW, BLK), 0)
    oh = (gids == jnp.broadcast_to(ids, (GW, BLK))).astype(jnp.float32)
    acc[...] += jnp.sum(oh, axis=1, keepdims=True)

    @pl.when(i == NB - 1)
    def _():
        cts = acc[...]
        ii = lax.broadcasted_iota(jnp.int32, (GW, GW), 0)
        jj = lax.broadcasted_iota(jnp.int32, (GW, GW), 1)
        tri = (jj < ii).astype(jnp.float32)
        starts_ref[...] = jnp.dot(
            tri, cts, preferred_element_type=jnp.float32).astype(jnp.int32)
        counts_ref[...] = cts


def _t3_call(batch_r):
    return pl.pallas_call(
        _t3_body,
        grid=(NB,),
        in_specs=[pl.BlockSpec((1, 1, BLK), lambda i: (i, 0, 0))],
        out_specs=[
            pl.BlockSpec((GW, 1), lambda i: (0, 0)),
            pl.BlockSpec((GW, 1), lambda i: (0, 0)),
        ],
        out_shape=[
            jax.ShapeDtypeStruct((GW, 1), jnp.int32),
            jax.ShapeDtypeStruct((GW, 1), jnp.float32),
        ],
        scratch_shapes=[pltpu.VMEM((GW, 1), jnp.float32)],
    )(batch_r)


def _t4_body(sums_ref, maxs_ref, counts_ref, wp1_ref, bp1_ref, wp2_ref,
             bp2_ref, out_ref):
    cnt = counts_ref[...][:G]
    mean = sums_ref[...] / jnp.maximum(cnt, 1.0)
    hc = jnp.concatenate([mean, maxs_ref[...]], axis=1)
    h = jnp.maximum(
        jnp.dot(hc, wp1_ref[...], preferred_element_type=jnp.float32)
        + bp1_ref[...], 0.0)
    out_ref[...] = (jnp.dot(h, wp2_ref[...], preferred_element_type=jnp.float32)
                    + bp2_ref[...])


def _t4_call(sums, maxs, counts, Wp1, bp1, Wp2, bp2):
    return pl.pallas_call(
        _t4_body,
        in_specs=[
            pl.BlockSpec((G, HID), lambda: (0, 0)),
            pl.BlockSpec((G, HID), lambda: (0, 0)),
            pl.BlockSpec((GW, 1), lambda: (0, 0)),
            pl.BlockSpec((2 * HID, HID), lambda: (0, 0)),
            pl.BlockSpec((1, HID), lambda: (0, 0)),
            pl.BlockSpec((HID, 3), lambda: (0, 0)),
            pl.BlockSpec((1, 3), lambda: (0, 0)),
        ],
        out_specs=pl.BlockSpec((G, 3), lambda: (0, 0)),
        out_shape=jax.ShapeDtypeStruct((G, 3), jnp.float32),
    )(sums, maxs, counts, Wp1, bp1, Wp2, bp2)


def kernel(x, edge_index, batch, W1, b1, W2, b2, W3, b3, g1, be1, g2, be2,
           g3, be3, Wp1, bp1, Wp2, bp2):
    e = edge_index.shape[1]
    pad = E_PAD - e
    src_p = jnp.concatenate([edge_index[0], jnp.zeros((pad,), jnp.int32)])
    dst_p = jnp.concatenate(
        [edge_index[1], jnp.full((pad,), DUMP_FILL, jnp.int32)])

    zeros_deg = jnp.zeros((DEG_PAD,), jnp.float32)
    zeros_acc = jnp.zeros((ROWS_PT, HID), jnp.float32)

    degp = _deg_call(dst_p, zeros_deg)
    starts_c, counts_c = _t3_call(batch.reshape(NB, 1, BLK))
    starts1d = starts_c.reshape(GW)

    hs, dinv = _t1_call(x, W1, degp)

    layers = ((b1, g1, be1, W2), (b2, g2, be2, W3), (b3, g3, be3, None))
    h3 = None
    for b, g, be, Wn in layers:
        agg = _agg_call(src_p, dst_p, hs, zeros_acc)
        agg3 = agg.reshape(NC, ACC_ROWS, HID)
        z, st = _t2a_call(agg3, hs, dinv, b.reshape(1, HID))
        if Wn is not None:
            hs = _t2b_call(z, st, dinv, g.reshape(1, HID), be.reshape(1, HID),
                           Wn)
        else:
            h3 = _t2b3_call(z, st, g.reshape(1, HID), be.reshape(1, HID))

    sums, maxs = _pool_call(h3, starts1d)
    return _t4_call(sums, maxs, counts_c, Wp1, bp1.reshape(1, HID), Wp2,
                    bp2.reshape(1, 3))
